# Initial kernel scaffold; baseline (speedup 1.0000x reference)
import jax
import jax.numpy as jnp
from jax.experimental import pallas as pl


def kernel(context, embeddings):
    raise NotImplementedError("write your pallas kernel here")


# speedup vs baseline: 9.8603x; 9.8603x over previous
import jax
import jax.numpy as jnp
import numpy as np
from jax import lax
from jax.experimental import pallas as pl
from jax.experimental.pallas import tpu as pltpu
from jax.experimental.pallas import tpu_sc as plsc

_K = 12
_N_NEG = 128
_TAU = 0.07
_B, _T, _D = 8, 512, 128
_TP = _T - _K
_R = _TP * _B
_V = _B * _T
_EPS = 1e-8
_INV_TAU = 1.0 / _TAU


def _threefry2x32(k1, k2, x0, x1):
    def rotl(x, d):
        return ((x << np.uint32(d)) | (x >> np.uint32(32 - d))).astype(np.uint32)

    ks0, ks1 = np.uint32(k1), np.uint32(k2)
    ks2 = np.uint32(ks0 ^ ks1 ^ np.uint32(0x1BD11BDA))
    x0 = (x0 + ks0).astype(np.uint32)
    x1 = (x1 + ks1).astype(np.uint32)
    sched = [(ks1, ks2), (ks2, ks0), (ks0, ks1), (ks1, ks2), (ks2, ks0)]
    rots = [(13, 15, 26, 6), (17, 29, 16, 24)]
    for i in range(5):
        for r in rots[i % 2]:
            x0 = (x0 + x1).astype(np.uint32)
            x1 = rotl(x1, r)
            x1 = (x0 ^ x1).astype(np.uint32)
        a, b = sched[i]
        x0 = (x0 + a).astype(np.uint32)
        x1 = (x1 + b + np.uint32(i + 1)).astype(np.uint32)
    return x0, x1


def _make_neg_idx() -> np.ndarray:
    b1, b2 = _threefry2x32(np.uint32(0), np.uint32(42),
                           np.zeros(2, np.uint32), np.arange(2, dtype=np.uint32))
    size = _TP * _B * _N_NEG
    o1, o2 = _threefry2x32(b1[1], b2[1],
                           np.zeros(size, np.uint32), np.arange(size, dtype=np.uint32))
    bits = o1 ^ o2
    return (bits % np.uint32(_V)).astype(np.int32).reshape(_R, _N_NEG)


_IDX = _make_neg_idx()


_BM = 200
_BN = 2048


def _sim_body(c_ref, z_ref, out_ref):
    c = c_ref[...]
    z = z_ref[...]
    na = jnp.sqrt(jnp.sum(c * c, axis=1, keepdims=True))
    nb = jnp.sqrt(jnp.sum(z * z, axis=1, keepdims=True))
    d = lax.dot_general(c, z, (((1,), (1,)), ((), ())),
                        preferred_element_type=jnp.float32)
    denom = jnp.maximum(na * nb.reshape(1, _BN), _EPS)
    out_ref[...] = d / denom * _INV_TAU


def _similarity(c2, flat):
    return pl.pallas_call(
        _sim_body,
        grid=(_R // _BM, _V // _BN),
        in_specs=[
            pl.BlockSpec((_BM, _D), lambda i, j: (i, 0)),
            pl.BlockSpec((_BN, _D), lambda i, j: (j, 0)),
        ],
        out_specs=pl.BlockSpec((_BM, _BN), lambda i, j: (i, j)),
        out_shape=jax.ShapeDtypeStruct((_R, _V), jnp.float32),
    )(c2, flat)


_NW = 32
_CHUNK = 8
_NCH = _R // _CHUNK


def _gather_body(s_hbm, idx_hbm, out_hbm, s_buf, idx_buf, g_buf):
    wid = lax.axis_index("s") * 2 + lax.axis_index("c")
    nk = (_NCH - wid + _NW - 1) // _NW

    def chunk(k, carry):
        c = wid + k * _NW
        pltpu.sync_copy(s_hbm.at[pl.ds(c * _CHUNK * _V, _CHUNK * _V)], s_buf)
        pltpu.sync_copy(
            idx_hbm.at[pl.ds(c * _CHUNK * _N_NEG, _CHUNK * _N_NEG)], idx_buf)
        for i in range(_CHUNK):
            for j in range(_N_NEG // 16):
                cols = idx_buf[pl.ds(i * _N_NEG + j * 16, 16)] + i * _V
                g_buf[pl.ds(i * _N_NEG + j * 16, 16)] = plsc.load_gather(
                    s_buf, [cols])
        pltpu.sync_copy(
            g_buf, out_hbm.at[pl.ds(c * _CHUNK * _N_NEG, _CHUNK * _N_NEG)])
        return carry

    lax.fori_loop(0, nk, chunk, 0)


def _gather(s, idx):
    out = pl.kernel(
        _gather_body,
        mesh=plsc.VectorSubcoreMesh(core_axis_name="c", subcore_axis_name="s"),
        compiler_params=pltpu.CompilerParams(needs_layout_passes=False),
        out_type=jax.ShapeDtypeStruct((_R * _N_NEG,), jnp.float32),
        scratch_types=[
            pltpu.VMEM((_CHUNK * _V,), jnp.float32),
            pltpu.VMEM((_CHUNK * _N_NEG,), jnp.int32),
            pltpu.VMEM((_CHUNK * _N_NEG,), jnp.float32),
        ],
    )(s.reshape(_R * _V), idx.reshape(_R * _N_NEG))
    return out.reshape(_R, _N_NEG)


def _loss_body(c_ref, zp_ref, g_ref, out_ref):
    c = c_ref[...]
    z = zp_ref[...]
    g = g_ref[...]
    na = jnp.sqrt(jnp.sum(c * c, axis=1, keepdims=True))
    nb = jnp.sqrt(jnp.sum(z * z, axis=1, keepdims=True))
    dot = jnp.sum(c * z, axis=1, keepdims=True)
    pos = dot / jnp.maximum(na * nb, _EPS) * _INV_TAU
    m = jnp.maximum(jnp.max(g, axis=1, keepdims=True), pos)
    se = jnp.exp(pos - m) + jnp.sum(jnp.exp(g - m), axis=1, keepdims=True)
    out_ref[0, 0] = jnp.mean(m + jnp.log(se) - pos)


def _loss(c2, zp2, g):
    res = pl.pallas_call(
        _loss_body,
        in_specs=[
            pl.BlockSpec((_R, _D), lambda: (0, 0)),
            pl.BlockSpec((_R, _D), lambda: (0, 0)),
            pl.BlockSpec((_R, _N_NEG), lambda: (0, 0)),
        ],
        out_specs=pl.BlockSpec(memory_space=pltpu.SMEM),
        out_shape=jax.ShapeDtypeStruct((1, 1), jnp.float32),
    )(c2, zp2, g)
    return res[0, 0]


def kernel(context, embeddings):
    c2 = jnp.transpose(context[:, :_TP, :], (1, 0, 2)).reshape(_R, _D)
    zp2 = jnp.transpose(embeddings[:, _K:, :], (1, 0, 2)).reshape(_R, _D)
    flat = embeddings.reshape(_V, _D)
    s = _similarity(c2, flat)
    g = _gather(s, jnp.asarray(_IDX))
    return _loss(c2, zp2, g)


# speedup vs baseline: 13.4740x; 1.3665x over previous
import jax
import jax.numpy as jnp
import numpy as np
from jax import lax
from jax.experimental import pallas as pl
from jax.experimental.pallas import tpu as pltpu
from jax.experimental.pallas import tpu_sc as plsc

_K = 12
_N_NEG = 128
_TAU = 0.07
_B, _T, _D = 8, 512, 128
_TP = _T - _K
_R = _TP * _B
_V = _B * _T
_EPS = 1e-8
_INV_TAU = 1.0 / _TAU


def _threefry2x32(k1, k2, x0, x1):
    def rotl(x, d):
        return ((x << np.uint32(d)) | (x >> np.uint32(32 - d))).astype(np.uint32)

    ks0, ks1 = np.uint32(k1), np.uint32(k2)
    ks2 = np.uint32(ks0 ^ ks1 ^ np.uint32(0x1BD11BDA))
    x0 = (x0 + ks0).astype(np.uint32)
    x1 = (x1 + ks1).astype(np.uint32)
    sched = [(ks1, ks2), (ks2, ks0), (ks0, ks1), (ks1, ks2), (ks2, ks0)]
    rots = [(13, 15, 26, 6), (17, 29, 16, 24)]
    for i in range(5):
        for r in rots[i % 2]:
            x0 = (x0 + x1).astype(np.uint32)
            x1 = rotl(x1, r)
            x1 = (x0 ^ x1).astype(np.uint32)
        a, b = sched[i]
        x0 = (x0 + a).astype(np.uint32)
        x1 = (x1 + b + np.uint32(i + 1)).astype(np.uint32)
    return x0, x1


def _make_neg_idx() -> np.ndarray:
    b1, b2 = _threefry2x32(np.uint32(0), np.uint32(42),
                           np.zeros(2, np.uint32), np.arange(2, dtype=np.uint32))
    size = _TP * _B * _N_NEG
    o1, o2 = _threefry2x32(b1[1], b2[1],
                           np.zeros(size, np.uint32), np.arange(size, dtype=np.uint32))
    bits = o1 ^ o2
    return (bits % np.uint32(_V)).astype(np.int32).reshape(_R, _N_NEG)


_IDX = _make_neg_idx()


_BM = 200
_BN = 2048


def _sim_body(c_ref, z_ref, out_ref):
    c = c_ref[...]
    z = z_ref[...]
    na = jnp.sqrt(jnp.sum(c * c, axis=1, keepdims=True))
    nb = jnp.sqrt(jnp.sum(z * z, axis=1, keepdims=True))
    d = lax.dot_general(c, z, (((1,), (1,)), ((), ())),
                        preferred_element_type=jnp.float32)
    denom = jnp.maximum(na * nb.reshape(1, _BN), _EPS)
    out_ref[...] = d / denom * _INV_TAU


def _similarity(c2, flat):
    return pl.pallas_call(
        _sim_body,
        grid=(_R // _BM, _V // _BN),
        in_specs=[
            pl.BlockSpec((_BM, _D), lambda i, j: (i, 0)),
            pl.BlockSpec((_BN, _D), lambda i, j: (j, 0)),
        ],
        out_specs=pl.BlockSpec((_BM, _BN), lambda i, j: (i, j)),
        out_shape=jax.ShapeDtypeStruct((_R, _V), jnp.float32),
    )(c2, flat)


_NW = 32
_CHUNK = 8
_NCH = _R // _CHUNK


def _gather_body(s_hbm, idx_hbm, out_hbm, s_buf, idx_buf, g_buf):
    wid = lax.axis_index("s") * 2 + lax.axis_index("c")
    nk = (_NCH - wid + _NW - 1) // _NW

    def chunk(k, carry):
        base = (wid + k * _NW) * _CHUNK
        pltpu.sync_copy(s_hbm.at[pl.ds(base, _CHUNK)], s_buf)
        pltpu.sync_copy(idx_hbm.at[pl.ds(base, _CHUNK)], idx_buf)
        for i in range(_CHUNK):
            rows = jnp.full((16,), i, jnp.int32)
            for j in range(_N_NEG // 16):
                cols = idx_buf[i, pl.ds(j * 16, 16)]
                g_buf[i, pl.ds(j * 16, 16)] = plsc.load_gather(
                    s_buf, [rows, cols])
        pltpu.sync_copy(g_buf, out_hbm.at[pl.ds(base, _CHUNK)])
        return carry

    lax.fori_loop(0, nk, chunk, 0)


def _gather(s, idx):
    return pl.kernel(
        _gather_body,
        mesh=plsc.VectorSubcoreMesh(core_axis_name="c", subcore_axis_name="s"),
        compiler_params=pltpu.CompilerParams(needs_layout_passes=False),
        out_type=jax.ShapeDtypeStruct((_R, _N_NEG), jnp.float32),
        scratch_types=[
            pltpu.VMEM((_CHUNK, _V), jnp.float32),
            pltpu.VMEM((_CHUNK, _N_NEG), jnp.int32),
            pltpu.VMEM((_CHUNK, _N_NEG), jnp.float32),
        ],
    )(s, idx)


def _loss_body(c_ref, zp_ref, g_ref, out_ref):
    c = c_ref[...]
    z = zp_ref[...]
    g = g_ref[...]
    na = jnp.sqrt(jnp.sum(c * c, axis=1, keepdims=True))
    nb = jnp.sqrt(jnp.sum(z * z, axis=1, keepdims=True))
    dot = jnp.sum(c * z, axis=1, keepdims=True)
    pos = dot / jnp.maximum(na * nb, _EPS) * _INV_TAU
    m = jnp.maximum(jnp.max(g, axis=1, keepdims=True), pos)
    se = jnp.exp(pos - m) + jnp.sum(jnp.exp(g - m), axis=1, keepdims=True)
    out_ref[0, 0] = jnp.mean(m + jnp.log(se) - pos)


def _loss(c2, zp2, g):
    res = pl.pallas_call(
        _loss_body,
        in_specs=[
            pl.BlockSpec((_R, _D), lambda: (0, 0)),
            pl.BlockSpec((_R, _D), lambda: (0, 0)),
            pl.BlockSpec((_R, _N_NEG), lambda: (0, 0)),
        ],
        out_specs=pl.BlockSpec(memory_space=pltpu.SMEM),
        out_shape=jax.ShapeDtypeStruct((1, 1), jnp.float32),
    )(c2, zp2, g)
    return res[0, 0]


def kernel(context, embeddings):
    c2 = jnp.transpose(context[:, :_TP, :], (1, 0, 2)).reshape(_R, _D)
    zp2 = jnp.transpose(embeddings[:, _K:, :], (1, 0, 2)).reshape(_R, _D)
    flat = embeddings.reshape(_V, _D)
    s = _similarity(c2, flat)
    g = _gather(s, jnp.asarray(_IDX))
    return _loss(c2, zp2, g)


# speedup vs baseline: 15.0738x; 1.1187x over previous
import jax
import jax.numpy as jnp
import numpy as np
from jax import lax
from jax.experimental import pallas as pl
from jax.experimental.pallas import tpu as pltpu
from jax.experimental.pallas import tpu_sc as plsc

_K = 12
_N_NEG = 128
_TAU = 0.07
_B, _T, _D = 8, 512, 128
_TP = _T - _K
_R = _TP * _B
_V = _B * _T
_EPS = 1e-8
_INV_TAU = 1.0 / _TAU


def _threefry2x32(k1, k2, x0, x1):
    def rotl(x, d):
        return ((x << np.uint32(d)) | (x >> np.uint32(32 - d))).astype(np.uint32)

    ks0, ks1 = np.uint32(k1), np.uint32(k2)
    ks2 = np.uint32(ks0 ^ ks1 ^ np.uint32(0x1BD11BDA))
    x0 = (x0 + ks0).astype(np.uint32)
    x1 = (x1 + ks1).astype(np.uint32)
    sched = [(ks1, ks2), (ks2, ks0), (ks0, ks1), (ks1, ks2), (ks2, ks0)]
    rots = [(13, 15, 26, 6), (17, 29, 16, 24)]
    for i in range(5):
        for r in rots[i % 2]:
            x0 = (x0 + x1).astype(np.uint32)
            x1 = rotl(x1, r)
            x1 = (x0 ^ x1).astype(np.uint32)
        a, b = sched[i]
        x0 = (x0 + a).astype(np.uint32)
        x1 = (x1 + b + np.uint32(i + 1)).astype(np.uint32)
    return x0, x1


def _make_neg_idx() -> np.ndarray:
    b1, b2 = _threefry2x32(np.uint32(0), np.uint32(42),
                           np.zeros(2, np.uint32), np.arange(2, dtype=np.uint32))
    size = _TP * _B * _N_NEG
    o1, o2 = _threefry2x32(b1[1], b2[1],
                           np.zeros(size, np.uint32), np.arange(size, dtype=np.uint32))
    bits = o1 ^ o2
    return (bits % np.uint32(_V)).astype(np.int32).reshape(_R, _N_NEG)


_IDX = _make_neg_idx()


_BM = 200
_BN = 2048


def _sim_body(c_ref, z_ref, out_ref):
    c = c_ref[...]
    z = z_ref[...]
    cn = c * (_INV_TAU / jnp.maximum(
        jnp.sqrt(jnp.sum(c * c, axis=1, keepdims=True)), 1e-6))
    zn = z / jnp.maximum(
        jnp.sqrt(jnp.sum(z * z, axis=1, keepdims=True)), 1e-6)
    out_ref[...] = lax.dot_general(cn, zn, (((1,), (1,)), ((), ())),
                                   preferred_element_type=jnp.float32)


def _similarity(c2, flat):
    return pl.pallas_call(
        _sim_body,
        grid=(_R // _BM, _V // _BN),
        in_specs=[
            pl.BlockSpec((_BM, _D), lambda i, j: (i, 0)),
            pl.BlockSpec((_BN, _D), lambda i, j: (j, 0)),
        ],
        out_specs=pl.BlockSpec((_BM, _BN), lambda i, j: (i, j)),
        out_shape=jax.ShapeDtypeStruct((_R, _V), jnp.float32),
    )(c2, flat)


_NW = 32
_CHUNK = 8
_NCH = _R // _CHUNK


_SLOTS = (_NCH + _NW - 1) // _NW


def _gather_body(s_hbm, idx_hbm, out_hbm, s_buf, idx_buf, g_buf,
                 sem_s0, sem_s1, sem_i0, sem_i1, sem_o0, sem_o1):
    wid = lax.axis_index("s") * 2 + lax.axis_index("c")
    sem_s, sem_i, sem_o = (sem_s0, sem_s1), (sem_i0, sem_i1), (sem_o0, sem_o1)

    def in_copies(k):
        c = wid + k * _NW
        slot = k % 2
        base = c * _CHUNK
        return (
            c,
            pltpu.make_async_copy(s_hbm.at[pl.ds(base, _CHUNK)],
                                  s_buf.at[slot], sem_s[slot]),
            pltpu.make_async_copy(idx_hbm.at[pl.ds(base, _CHUNK)],
                                  idx_buf.at[slot], sem_i[slot]),
        )

    def out_copy(k):
        c = wid + k * _NW
        slot = k % 2
        return c, pltpu.make_async_copy(
            g_buf.at[slot], out_hbm.at[pl.ds(c * _CHUNK, _CHUNK)], sem_o[slot])

    c0, cp_s, cp_i = in_copies(0)

    @pl.when(c0 < _NCH)
    def _():
        cp_s.start()
        cp_i.start()

    for k in range(_SLOTS):
        slot = k % 2
        if k + 1 < _SLOTS:
            cn, cp_sn, cp_in = in_copies(k + 1)

            @pl.when(cn < _NCH)
            def _(cp_sn=cp_sn, cp_in=cp_in):
                cp_sn.start()
                cp_in.start()

        c, cp_s, cp_i = in_copies(k)

        @pl.when(c < _NCH)
        def _(k=k, slot=slot, c=c, cp_s=cp_s, cp_i=cp_i):
            cp_s.wait()
            cp_i.wait()
            if k >= 2:
                _, cp_prev = out_copy(k - 2)
                cp_prev.wait()
            for i in range(_CHUNK):
                rows = jnp.full((16,), i, jnp.int32)
                for j in range(_N_NEG // 16):
                    cols = idx_buf[slot, i, pl.ds(j * 16, 16)]
                    g_buf[slot, i, pl.ds(j * 16, 16)] = plsc.load_gather(
                        s_buf.at[slot], [rows, cols])
            _, cp_o = out_copy(k)
            cp_o.start()

    for k in (_SLOTS - 2, _SLOTS - 1):
        c, cp_o = out_copy(k)

        @pl.when(c < _NCH)
        def _(cp_o=cp_o):
            cp_o.wait()


def _gather(s, idx):
    return pl.kernel(
        _gather_body,
        mesh=plsc.VectorSubcoreMesh(core_axis_name="c", subcore_axis_name="s"),
        compiler_params=pltpu.CompilerParams(needs_layout_passes=False),
        out_type=jax.ShapeDtypeStruct((_R, _N_NEG), jnp.float32),
        scratch_types=[
            pltpu.VMEM((2, _CHUNK, _V), jnp.float32),
            pltpu.VMEM((2, _CHUNK, _N_NEG), jnp.int32),
            pltpu.VMEM((2, _CHUNK, _N_NEG), jnp.float32),
            pltpu.SemaphoreType.DMA,
            pltpu.SemaphoreType.DMA,
            pltpu.SemaphoreType.DMA,
            pltpu.SemaphoreType.DMA,
            pltpu.SemaphoreType.DMA,
            pltpu.SemaphoreType.DMA,
        ],
    )(s, idx)


def _loss_body(c_ref, zp_ref, g_ref, out_ref):
    c = c_ref[...]
    z = zp_ref[...]
    g = g_ref[...]
    na = jnp.sqrt(jnp.sum(c * c, axis=1, keepdims=True))
    nb = jnp.sqrt(jnp.sum(z * z, axis=1, keepdims=True))
    dot = jnp.sum(c * z, axis=1, keepdims=True)
    pos = dot / jnp.maximum(na * nb, _EPS) * _INV_TAU
    m = jnp.maximum(jnp.max(g, axis=1, keepdims=True), pos)
    se = jnp.exp(pos - m) + jnp.sum(jnp.exp(g - m), axis=1, keepdims=True)
    out_ref[0, 0] = jnp.mean(m + jnp.log(se) - pos)


def _loss(c2, zp2, g):
    res = pl.pallas_call(
        _loss_body,
        in_specs=[
            pl.BlockSpec((_R, _D), lambda: (0, 0)),
            pl.BlockSpec((_R, _D), lambda: (0, 0)),
            pl.BlockSpec((_R, _N_NEG), lambda: (0, 0)),
        ],
        out_specs=pl.BlockSpec(memory_space=pltpu.SMEM),
        out_shape=jax.ShapeDtypeStruct((1, 1), jnp.float32),
    )(c2, zp2, g)
    return res[0, 0]


def kernel(context, embeddings):
    c2 = jnp.transpose(context[:, :_TP, :], (1, 0, 2)).reshape(_R, _D)
    zp2 = jnp.transpose(embeddings[:, _K:, :], (1, 0, 2)).reshape(_R, _D)
    flat = embeddings.reshape(_V, _D)
    s = _similarity(c2, flat)
    g = _gather(s, jnp.asarray(_IDX))
    return _loss(c2, zp2, g)
